# baseline (speedup 1.0000x reference)
import functools

import jax
import jax.numpy as jnp
from jax import lax
from jax.experimental import pallas as pl
from jax.experimental.pallas import tpu as pltpu
from jax.experimental.pallas import tpu_sc as plsc

N = 10000
E = 160000
M = 256
GAMMA = 0.8
MAX_ITER = 30
EPS_F = 1e-12

K = MAX_ITER - 1
NC, NS, LANES = 2, 16, 16
NW = NC * NS
FPW = 4
NSLICE = M // FPW
SLICE_WORDS = N * FPW
CHUNK = 2000
NCHUNK = E // CHUNK
GROUPS = CHUNK // LANES


def _sc_mesh():
    return plsc.VectorSubcoreMesh(core_axis_name="c", subcore_axis_name="s",
                                  num_cores=NC, num_subcores=NS)


def _scan_edges(rows_hbm, cols_hbm, w_hbm, rb, cb, wb, zb, ub, sem):
    def chunk_body(ci, _):
        off = ci * CHUNK
        pltpu.async_copy(rows_hbm.at[pl.ds(off, CHUNK)], rb, sem).wait()
        pltpu.async_copy(cols_hbm.at[pl.ds(off, CHUNK)], cb, sem).wait()
        pltpu.async_copy(w_hbm.at[pl.ds(off, CHUNK)], wb, sem).wait()

        def group_body(gi, _):
            b = gi * LANES
            r = rb[pl.ds(b, LANES)]
            c = cb[pl.ds(b, LANES)]
            w = wb[pl.ds(b, LANES)]
            r4 = r * FPW
            c4 = c * FPW
            for j in range(FPW):
                rj = r4 if j == 0 else r4 + j
                cj = c4 if j == 0 else c4 + j
                zj = plsc.load_gather(zb, [rj])
                plsc.addupdate_scatter(ub, [cj], zj * w)
            return 0

        lax.fori_loop(0, GROUPS, group_body, 0)
        return 0

    lax.fori_loop(0, NCHUNK, chunk_body, 0)


def _zero_buf(buf):
    zeros = jnp.zeros((LANES,), jnp.float32)

    def body(i, _):
        buf[pl.ds(i * LANES, LANES)] = zeros
        return 0

    lax.fori_loop(0, SLICE_WORDS // LANES, body, 0)


@functools.partial(
    pl.kernel,
    out_type=jax.ShapeDtypeStruct((K, NSLICE, SLICE_WORDS), jnp.float32),
    mesh=_sc_mesh(),
    compiler_params=pltpu.CompilerParams(needs_layout_passes=False),
    scratch_types=[
        pltpu.VMEM((SLICE_WORDS,), jnp.float32),
        pltpu.VMEM((SLICE_WORDS,), jnp.float32),
        pltpu.VMEM((CHUNK,), jnp.int32),
        pltpu.VMEM((CHUNK,), jnp.int32),
        pltpu.VMEM((CHUNK,), jnp.float32),
        pltpu.SemaphoreType.DMA,
    ],
)
def _sc_propagate(x_hbm, rows_hbm, cols_hbm, w_hbm, p_hbm,
                  zb, ub, rb, cb, wb, sem):
    wid = lax.axis_index("s") * NC + lax.axis_index("c")

    for slice_i in range(NSLICE // NW):
        s = slice_i * NW + wid
        pltpu.async_copy(x_hbm.at[s], zb, sem).wait()

        def k_body(k, _):
            _zero_buf(ub)
            _scan_edges(rows_hbm, cols_hbm, w_hbm, rb, cb, wb, zb, ub, sem)
            pltpu.async_copy(ub, p_hbm.at[k, s], sem).wait()
            pltpu.async_copy(p_hbm.at[k, s], zb, sem).wait()
            return 0

        lax.fori_loop(0, K, k_body, 0)


def _cchain_body(f_ref, out_ref, base_ref, c_ref):
    k = pl.program_id(0)

    @pl.when(k == 0)
    def _():
        ff = lax.dot_general(f_ref[...], f_ref[...],
                             (((0,), (0,)), ((), ())),
                             preferred_element_type=jnp.float32)
        nrm = jnp.sqrt(jnp.sum(ff * ff))
        base_ref[...] = (GAMMA / (nrm + EPS_F)) * ff
        c_ref[...] = base_ref[...]

    @pl.when(k > 0)
    def _():
        c_ref[...] = jnp.dot(c_ref[...], base_ref[...],
                             preferred_element_type=jnp.float32)

    out_ref[0] = c_ref[...]


def _cchain(F):
    return pl.pallas_call(
        _cchain_body,
        grid=(K,),
        in_specs=[pl.BlockSpec((M, M), lambda k: (0, 0))],
        out_specs=pl.BlockSpec((1, M, M), lambda k: (k, 0, 0)),
        out_shape=jax.ShapeDtypeStruct((K, M, M), jnp.float32),
        scratch_shapes=[pltpu.VMEM((M, M), jnp.float32),
                        pltpu.VMEM((M, M), jnp.float32)],
    )(F)


BN = 2000


def _accum_body(x_ref, p_ref, c_ref, out_ref, acc_ref):
    k = pl.program_id(1)

    @pl.when(k == 0)
    def _():
        acc_ref[...] = x_ref[...]

    @pl.when(k > 0)
    def _():
        acc_ref[...] += jnp.dot(p_ref[0], c_ref[0],
                                preferred_element_type=jnp.float32)

    @pl.when(k == MAX_ITER - 1)
    def _():
        out_ref[...] = acc_ref[...]


def _accumulate(X, P, C):
    nb = N // BN
    return pl.pallas_call(
        _accum_body,
        grid=(nb, MAX_ITER),
        in_specs=[
            pl.BlockSpec((BN, M), lambda b, k: (b, 0)),
            pl.BlockSpec((1, BN, M), lambda b, k: (jnp.maximum(k - 1, 0), b, 0)),
            pl.BlockSpec((1, M, M), lambda b, k: (jnp.maximum(k - 1, 0), 0, 0)),
        ],
        out_specs=pl.BlockSpec((BN, M), lambda b, k: (b, 0)),
        out_shape=jax.ShapeDtypeStruct((N, M), jnp.float32),
        scratch_shapes=[pltpu.VMEM((BN, M), jnp.float32)],
    )(X, P, C)


def kernel(X, edge_index, edge_weight, F):
    rows = edge_index[0].astype(jnp.int32)
    cols = edge_index[1].astype(jnp.int32)
    w = edge_weight.astype(jnp.float32)

    x_sl = X.reshape(N, NSLICE, FPW).transpose(1, 0, 2).reshape(NSLICE, SLICE_WORDS)
    p_sl = _sc_propagate(x_sl, rows, cols, w)
    P = p_sl.reshape(K, NSLICE, N, FPW).transpose(0, 2, 1, 3).reshape(K, N, M)

    C = _cchain(F)
    return _accumulate(X, P, C)


# speedup vs baseline: 3.0289x; 3.0289x over previous
import functools

import jax
import jax.numpy as jnp
from jax import lax
from jax.experimental import pallas as pl
from jax.experimental.pallas import tpu as pltpu
from jax.experimental.pallas import tpu_sc as plsc

N = 10000
E = 160000
M = 256
GAMMA = 0.8
MAX_ITER = 30
EPS_F = 1e-12

K = MAX_ITER - 1
NC, NS, LANES = 2, 16, 16
NW = NC * NS
FPW = 4
NSLICE = M // FPW
SLICE_WORDS = N * FPW
CHUNK = 4000
NCHUNK = E // CHUNK
CGROUPS = CHUNK // LANES
UNROLL = 8


def _sc_mesh():
    return plsc.VectorSubcoreMesh(core_axis_name="c", subcore_axis_name="s",
                                  num_cores=NC, num_subcores=NS)


def _issue_chunk(ci, rows_hbm, cols_hbm, w_hbm, rb, cb, wb, sem):
    off = ci * CHUNK
    pltpu.async_copy(rows_hbm.at[pl.ds(off, CHUNK)], rb, sem)
    pltpu.async_copy(cols_hbm.at[pl.ds(off, CHUNK)], cb, sem)
    pltpu.async_copy(w_hbm.at[pl.ds(off, CHUNK)], wb, sem)


def _wait_chunk(ci, rows_hbm, cols_hbm, w_hbm, rb, cb, wb, sem):
    off = ci * CHUNK
    pltpu.make_async_copy(rows_hbm.at[pl.ds(off, CHUNK)], rb, sem).wait()
    pltpu.make_async_copy(cols_hbm.at[pl.ds(off, CHUNK)], cb, sem).wait()
    pltpu.make_async_copy(w_hbm.at[pl.ds(off, CHUNK)], wb, sem).wait()


def _process_chunk(rb, cb, wb, src, dst):
    @plsc.parallel_loop(0, CGROUPS, unroll=UNROLL)
    def _group(gi):
        b0 = gi * LANES
        r4 = rb[pl.ds(b0, LANES)] * FPW
        c4 = cb[pl.ds(b0, LANES)] * FPW
        w = wb[pl.ds(b0, LANES)]
        for j in range(FPW):
            rj = r4 if j == 0 else r4 + j
            cj = c4 if j == 0 else c4 + j
            zj = plsc.load_gather(src, [rj])
            plsc.addupdate_scatter(dst, [cj], zj * w)


def _zero_buf(buf):
    zeros = jnp.zeros((LANES,), jnp.float32)

    @plsc.parallel_loop(0, SLICE_WORDS // LANES, unroll=UNROLL)
    def _z(i):
        buf[pl.ds(i * LANES, LANES)] = zeros


def _scan_edges(rows_hbm, cols_hbm, w_hbm, bufs_a, bufs_b, sem_a, sem_b,
                src, dst):
    _issue_chunk(0, rows_hbm, cols_hbm, w_hbm, *bufs_a, sem_a)

    def pair_body(i, _):
        cbase = i * 2
        for b, (bufs, sem, nbufs, nsem) in enumerate(
                ((bufs_a, sem_a, bufs_b, sem_b),
                 (bufs_b, sem_b, bufs_a, sem_a))):
            ci = cbase + b
            _wait_chunk(ci, rows_hbm, cols_hbm, w_hbm, *bufs, sem)

            @pl.when(ci + 1 < NCHUNK)
            def _():
                _issue_chunk(ci + 1, rows_hbm, cols_hbm, w_hbm, *nbufs, nsem)

            _process_chunk(*bufs, src, dst)
        return 0

    lax.fori_loop(0, NCHUNK // 2, pair_body, 0)


@functools.partial(
    pl.kernel,
    out_type=jax.ShapeDtypeStruct((K, NSLICE, SLICE_WORDS), jnp.float32),
    mesh=_sc_mesh(),
    compiler_params=pltpu.CompilerParams(needs_layout_passes=False),
    scratch_types=[
        pltpu.VMEM((SLICE_WORDS,), jnp.float32),
        pltpu.VMEM((SLICE_WORDS,), jnp.float32),
        pltpu.VMEM((CHUNK,), jnp.int32),
        pltpu.VMEM((CHUNK,), jnp.int32),
        pltpu.VMEM((CHUNK,), jnp.float32),
        pltpu.VMEM((CHUNK,), jnp.int32),
        pltpu.VMEM((CHUNK,), jnp.int32),
        pltpu.VMEM((CHUNK,), jnp.float32),
        pltpu.SemaphoreType.DMA,
        pltpu.SemaphoreType.DMA,
        pltpu.SemaphoreType.DMA,
    ],
)
def _sc_propagate(x_hbm, rows_hbm, cols_hbm, w_hbm, p_hbm,
                  ping, pong, ra, ca, wa, rb, cb, wb, sem_a, sem_b, sem_o):
    wid = lax.axis_index("s") * NC + lax.axis_index("c")
    bufs_a = (ra, ca, wa)
    bufs_b = (rb, cb, wb)

    def step(k, s, src, dst):
        @pl.when(k >= 2)
        def _():
            pltpu.make_async_copy(dst, p_hbm.at[k - 2, s], sem_o).wait()

        _zero_buf(dst)
        _scan_edges(rows_hbm, cols_hbm, w_hbm, bufs_a, bufs_b,
                    sem_a, sem_b, src, dst)
        pltpu.async_copy(dst, p_hbm.at[k, s], sem_o)

    for slice_i in range(NSLICE // NW):
        s = slice_i * NW + wid
        pltpu.async_copy(x_hbm.at[s], ping, sem_o).wait()

        def pair_body(i, _):
            step(2 * i, s, ping, pong)
            step(2 * i + 1, s, pong, ping)
            return 0

        lax.fori_loop(0, (K - 1) // 2, pair_body, 0)
        step(K - 1, s, ping, pong)

        for k in (K - 2, K - 1):
            dst = pong if k % 2 == 0 else ping
            pltpu.make_async_copy(dst, p_hbm.at[k, s], sem_o).wait()


def _cchain_body(f_ref, out_ref, base_ref, c_ref):
    k = pl.program_id(0)

    @pl.when(k == 0)
    def _():
        ff = lax.dot_general(f_ref[...], f_ref[...],
                             (((0,), (0,)), ((), ())),
                             preferred_element_type=jnp.float32)
        nrm = jnp.sqrt(jnp.sum(ff * ff))
        base_ref[...] = (GAMMA / (nrm + EPS_F)) * ff
        c_ref[...] = base_ref[...]

    @pl.when(k > 0)
    def _():
        c_ref[...] = jnp.dot(c_ref[...], base_ref[...],
                             preferred_element_type=jnp.float32)

    out_ref[0] = c_ref[...]


def _cchain(F):
    return pl.pallas_call(
        _cchain_body,
        grid=(K,),
        in_specs=[pl.BlockSpec((M, M), lambda k: (0, 0))],
        out_specs=pl.BlockSpec((1, M, M), lambda k: (k, 0, 0)),
        out_shape=jax.ShapeDtypeStruct((K, M, M), jnp.float32),
        scratch_shapes=[pltpu.VMEM((M, M), jnp.float32),
                        pltpu.VMEM((M, M), jnp.float32)],
    )(F)


BN = 2000


def _accum_body(x_ref, p_ref, c_ref, out_ref, acc_ref):
    k = pl.program_id(1)

    @pl.when(k == 0)
    def _():
        acc_ref[...] = x_ref[...]

    @pl.when(k > 0)
    def _():
        acc_ref[...] += jnp.dot(p_ref[0], c_ref[0],
                                preferred_element_type=jnp.float32)

    @pl.when(k == MAX_ITER - 1)
    def _():
        out_ref[...] = acc_ref[...]


def _accumulate(X, P, C):
    nb = N // BN
    return pl.pallas_call(
        _accum_body,
        grid=(nb, MAX_ITER),
        in_specs=[
            pl.BlockSpec((BN, M), lambda b, k: (b, 0)),
            pl.BlockSpec((1, BN, M), lambda b, k: (jnp.maximum(k - 1, 0), b, 0)),
            pl.BlockSpec((1, M, M), lambda b, k: (jnp.maximum(k - 1, 0), 0, 0)),
        ],
        out_specs=pl.BlockSpec((BN, M), lambda b, k: (b, 0)),
        out_shape=jax.ShapeDtypeStruct((N, M), jnp.float32),
        scratch_shapes=[pltpu.VMEM((BN, M), jnp.float32)],
    )(X, P, C)


def kernel(X, edge_index, edge_weight, F):
    rows = edge_index[0].astype(jnp.int32)
    cols = edge_index[1].astype(jnp.int32)
    w = edge_weight.astype(jnp.float32)

    x_sl = X.reshape(N, NSLICE, FPW).transpose(1, 0, 2).reshape(NSLICE, SLICE_WORDS)
    p_sl = _sc_propagate(x_sl, rows, cols, w)
    P = p_sl.reshape(K, NSLICE, N, FPW).transpose(0, 2, 1, 3).reshape(K, N, M)

    C = _cchain(F)
    return _accumulate(X, P, C)


# speedup vs baseline: 4.6456x; 1.5337x over previous
import functools

import jax
import jax.numpy as jnp
from jax import lax
from jax.experimental import pallas as pl
from jax.experimental.pallas import tpu as pltpu
from jax.experimental.pallas import tpu_sc as plsc

N = 10000
E = 160000
M = 256
GAMMA = 0.8
MAX_ITER = 30
EPS_F = 1e-12

K = MAX_ITER - 1
NC, NS, LANES = 2, 16, 16
NW = NC * NS
FPW = 4
NSLICE = M // FPW
SLICE_WORDS = N * FPW
CHUNK = 4000
NCHUNK = E // CHUNK
CGROUPS = CHUNK // LANES
UNROLL = 8


def _sc_mesh():
    return plsc.VectorSubcoreMesh(core_axis_name="c", subcore_axis_name="s",
                                  num_cores=NC, num_subcores=NS)


def _issue_chunk(ci, rows_hbm, cols_hbm, w_hbm, rb, cb, wb, sem):
    off = ci * CHUNK
    pltpu.async_copy(rows_hbm.at[pl.ds(off, CHUNK)], rb, sem)
    pltpu.async_copy(cols_hbm.at[pl.ds(off, CHUNK)], cb, sem)
    pltpu.async_copy(w_hbm.at[pl.ds(off, CHUNK)], wb, sem)


def _wait_chunk(ci, rows_hbm, cols_hbm, w_hbm, rb, cb, wb, sem):
    off = ci * CHUNK
    pltpu.make_async_copy(rows_hbm.at[pl.ds(off, CHUNK)], rb, sem).wait()
    pltpu.make_async_copy(cols_hbm.at[pl.ds(off, CHUNK)], cb, sem).wait()
    pltpu.make_async_copy(w_hbm.at[pl.ds(off, CHUNK)], wb, sem).wait()


def _process_chunk(rb, cb, wb, src, dst):
    @plsc.parallel_loop(0, CGROUPS, unroll=UNROLL)
    def _group(gi):
        b0 = gi * LANES
        r = rb[pl.ds(b0, LANES)]
        c = cb[pl.ds(b0, LANES)]
        w = wb[pl.ds(b0, LANES)]
        for j in range(FPW):
            rj = r if j == 0 else r + (j * N)
            cj = c if j == 0 else c + (j * N)
            zj = plsc.load_gather(src, [rj])
            plsc.addupdate_scatter(dst, [cj], zj * w)


def _zero_buf(buf):
    zeros = jnp.zeros((LANES,), jnp.float32)

    @plsc.parallel_loop(0, SLICE_WORDS // LANES, unroll=UNROLL)
    def _z(i):
        buf[pl.ds(i * LANES, LANES)] = zeros


def _scan_edges(rows_hbm, cols_hbm, w_hbm, bufs_a, bufs_b, sem_a, sem_b,
                src, dst):
    _issue_chunk(0, rows_hbm, cols_hbm, w_hbm, *bufs_a, sem_a)

    def pair_body(i, _):
        cbase = i * 2
        for b, (bufs, sem, nbufs, nsem) in enumerate(
                ((bufs_a, sem_a, bufs_b, sem_b),
                 (bufs_b, sem_b, bufs_a, sem_a))):
            ci = cbase + b
            _wait_chunk(ci, rows_hbm, cols_hbm, w_hbm, *bufs, sem)

            @pl.when(ci + 1 < NCHUNK)
            def _():
                _issue_chunk(ci + 1, rows_hbm, cols_hbm, w_hbm, *nbufs, nsem)

            _process_chunk(*bufs, src, dst)
        return 0

    lax.fori_loop(0, NCHUNK // 2, pair_body, 0)


@functools.partial(
    pl.kernel,
    out_type=jax.ShapeDtypeStruct((K, NSLICE, SLICE_WORDS), jnp.float32),
    mesh=_sc_mesh(),
    compiler_params=pltpu.CompilerParams(needs_layout_passes=False),
    scratch_types=[
        pltpu.VMEM((SLICE_WORDS,), jnp.float32),
        pltpu.VMEM((SLICE_WORDS,), jnp.float32),
        pltpu.VMEM((CHUNK,), jnp.int32),
        pltpu.VMEM((CHUNK,), jnp.int32),
        pltpu.VMEM((CHUNK,), jnp.float32),
        pltpu.VMEM((CHUNK,), jnp.int32),
        pltpu.VMEM((CHUNK,), jnp.int32),
        pltpu.VMEM((CHUNK,), jnp.float32),
        pltpu.SemaphoreType.DMA,
        pltpu.SemaphoreType.DMA,
        pltpu.SemaphoreType.DMA,
    ],
)
def _sc_propagate(x_hbm, rows_hbm, cols_hbm, w_hbm, p_hbm,
                  ping, pong, ra, ca, wa, rb, cb, wb, sem_a, sem_b, sem_o):
    wid = lax.axis_index("s") * NC + lax.axis_index("c")
    bufs_a = (ra, ca, wa)
    bufs_b = (rb, cb, wb)

    def step(k, s, src, dst):
        @pl.when(k >= 2)
        def _():
            pltpu.make_async_copy(dst, p_hbm.at[k - 2, s], sem_o).wait()

        _zero_buf(dst)
        _scan_edges(rows_hbm, cols_hbm, w_hbm, bufs_a, bufs_b,
                    sem_a, sem_b, src, dst)
        pltpu.async_copy(dst, p_hbm.at[k, s], sem_o)

    for slice_i in range(NSLICE // NW):
        s = slice_i * NW + wid
        pltpu.async_copy(x_hbm.at[s], ping, sem_o).wait()

        def pair_body(i, _):
            step(2 * i, s, ping, pong)
            step(2 * i + 1, s, pong, ping)
            return 0

        lax.fori_loop(0, (K - 1) // 2, pair_body, 0)
        step(K - 1, s, ping, pong)

        for k in (K - 2, K - 1):
            dst = pong if k % 2 == 0 else ping
            pltpu.make_async_copy(dst, p_hbm.at[k, s], sem_o).wait()


def _cchain_body(f_ref, out_ref, base_ref, c_ref):
    k = pl.program_id(0)

    @pl.when(k == 0)
    def _():
        ff = lax.dot_general(f_ref[...], f_ref[...],
                             (((0,), (0,)), ((), ())),
                             preferred_element_type=jnp.float32)
        nrm = jnp.sqrt(jnp.sum(ff * ff))
        base_ref[...] = (GAMMA / (nrm + EPS_F)) * ff
        c_ref[...] = base_ref[...]

    @pl.when(k > 0)
    def _():
        c_ref[...] = jnp.dot(c_ref[...], base_ref[...],
                             preferred_element_type=jnp.float32)

    out_ref[0] = c_ref[...]


def _cchain(F):
    return pl.pallas_call(
        _cchain_body,
        grid=(K,),
        in_specs=[pl.BlockSpec((M, M), lambda k: (0, 0))],
        out_specs=pl.BlockSpec((1, M, M), lambda k: (k, 0, 0)),
        out_shape=jax.ShapeDtypeStruct((K, M, M), jnp.float32),
        scratch_shapes=[pltpu.VMEM((M, M), jnp.float32),
                        pltpu.VMEM((M, M), jnp.float32)],
    )(F)


BN = 2000


def _accum_body(x_ref, p_ref, c_ref, out_ref, acc_ref):
    k = pl.program_id(1)

    @pl.when(k == 0)
    def _():
        acc_ref[...] = x_ref[...]

    @pl.when(k > 0)
    def _():
        acc_ref[...] += jnp.dot(p_ref[0], c_ref[0],
                                preferred_element_type=jnp.float32)

    @pl.when(k == MAX_ITER - 1)
    def _():
        out_ref[...] = acc_ref[...]


def _accumulate(X, P, C):
    nb = N // BN
    return pl.pallas_call(
        _accum_body,
        grid=(nb, MAX_ITER),
        in_specs=[
            pl.BlockSpec((BN, M), lambda b, k: (b, 0)),
            pl.BlockSpec((1, BN, M), lambda b, k: (jnp.maximum(k - 1, 0), b, 0)),
            pl.BlockSpec((1, M, M), lambda b, k: (jnp.maximum(k - 1, 0), 0, 0)),
        ],
        out_specs=pl.BlockSpec((BN, M), lambda b, k: (b, 0)),
        out_shape=jax.ShapeDtypeStruct((N, M), jnp.float32),
        scratch_shapes=[pltpu.VMEM((BN, M), jnp.float32)],
    )(X, P, C)


def kernel(X, edge_index, edge_weight, F):
    rows = edge_index[0].astype(jnp.int32)
    cols = edge_index[1].astype(jnp.int32)
    w = edge_weight.astype(jnp.float32)

    x_sl = X.reshape(N, NSLICE, FPW).transpose(1, 2, 0).reshape(NSLICE, SLICE_WORDS)
    p_sl = _sc_propagate(x_sl, rows, cols, w)
    P = p_sl.reshape(K, NSLICE, FPW, N).transpose(0, 3, 1, 2).reshape(K, N, M)

    C = _cchain(F)
    return _accumulate(X, P, C)


# speedup vs baseline: 5.1419x; 1.1068x over previous
import functools

import jax
import jax.numpy as jnp
from jax import lax
from jax.experimental import pallas as pl
from jax.experimental.pallas import tpu as pltpu
from jax.experimental.pallas import tpu_sc as plsc

N = 10000
E = 160000
M = 256
GAMMA = 0.8
MAX_ITER = 30
EPS_F = 1e-12

K = MAX_ITER - 1
NC, NS, LANES = 2, 16, 16
NW = NC * NS
FPW = 4
NSLICE = M // FPW
NPAD = 10240
SLICE_WORDS = NPAD * FPW
CHUNK = 4000
NCHUNK = E // CHUNK
CGROUPS = CHUNK // LANES
UNROLL = 10


def _sc_mesh():
    return plsc.VectorSubcoreMesh(core_axis_name="c", subcore_axis_name="s",
                                  num_cores=NC, num_subcores=NS)


def _issue_chunk(ci, rows_hbm, cols_hbm, w_hbm, rb, cb, wb, sem):
    off = ci * CHUNK
    pltpu.async_copy(rows_hbm.at[pl.ds(off, CHUNK)], rb, sem)
    pltpu.async_copy(cols_hbm.at[pl.ds(off, CHUNK)], cb, sem)
    pltpu.async_copy(w_hbm.at[pl.ds(off, CHUNK)], wb, sem)


def _wait_chunk(ci, rows_hbm, cols_hbm, w_hbm, rb, cb, wb, sem):
    off = ci * CHUNK
    pltpu.make_async_copy(rows_hbm.at[pl.ds(off, CHUNK)], rb, sem).wait()
    pltpu.make_async_copy(cols_hbm.at[pl.ds(off, CHUNK)], cb, sem).wait()
    pltpu.make_async_copy(w_hbm.at[pl.ds(off, CHUNK)], wb, sem).wait()


def _process_chunk(rb, cb, wb, src, dst):
    @plsc.parallel_loop(0, CGROUPS, unroll=UNROLL)
    def _group(gi):
        b0 = gi * LANES
        r = rb[pl.ds(b0, LANES)]
        c = cb[pl.ds(b0, LANES)]
        w = wb[pl.ds(b0, LANES)]
        for j in range(FPW):
            rj = r if j == 0 else r + (j * NPAD)
            cj = c if j == 0 else c + (j * NPAD)
            zj = plsc.load_gather(src, [rj])
            plsc.addupdate_scatter(dst, [cj], zj * w)


def _zero_buf(buf):
    zeros = jnp.zeros((LANES,), jnp.float32)

    @plsc.parallel_loop(0, SLICE_WORDS // LANES, unroll=UNROLL)
    def _z(i):
        buf[pl.ds(i * LANES, LANES)] = zeros


def _scan_edges(rows_hbm, cols_hbm, w_hbm, bufs_a, bufs_b, sem_a, sem_b,
                src, dst):
    _issue_chunk(0, rows_hbm, cols_hbm, w_hbm, *bufs_a, sem_a)

    def pair_body(i, _):
        cbase = i * 2
        for b, (bufs, sem, nbufs, nsem) in enumerate(
                ((bufs_a, sem_a, bufs_b, sem_b),
                 (bufs_b, sem_b, bufs_a, sem_a))):
            ci = cbase + b
            _wait_chunk(ci, rows_hbm, cols_hbm, w_hbm, *bufs, sem)

            @pl.when(ci + 1 < NCHUNK)
            def _():
                _issue_chunk(ci + 1, rows_hbm, cols_hbm, w_hbm, *nbufs, nsem)

            _process_chunk(*bufs, src, dst)
        return 0

    lax.fori_loop(0, NCHUNK // 2, pair_body, 0)


@functools.partial(
    pl.kernel,
    out_type=jax.ShapeDtypeStruct((K, NSLICE, SLICE_WORDS), jnp.float32),
    mesh=_sc_mesh(),
    compiler_params=pltpu.CompilerParams(needs_layout_passes=False),
    scratch_types=[
        pltpu.VMEM((SLICE_WORDS,), jnp.float32),
        pltpu.VMEM((SLICE_WORDS,), jnp.float32),
        pltpu.VMEM((CHUNK,), jnp.int32),
        pltpu.VMEM((CHUNK,), jnp.int32),
        pltpu.VMEM((CHUNK,), jnp.float32),
        pltpu.VMEM((CHUNK,), jnp.int32),
        pltpu.VMEM((CHUNK,), jnp.int32),
        pltpu.VMEM((CHUNK,), jnp.float32),
        pltpu.SemaphoreType.DMA,
        pltpu.SemaphoreType.DMA,
        pltpu.SemaphoreType.DMA,
    ],
)
def _sc_propagate(x_hbm, rows_hbm, cols_hbm, w_hbm, p_hbm,
                  ping, pong, ra, ca, wa, rb, cb, wb, sem_a, sem_b, sem_o):
    wid = lax.axis_index("s") * NC + lax.axis_index("c")
    bufs_a = (ra, ca, wa)
    bufs_b = (rb, cb, wb)

    def step(k, s, src, dst):
        @pl.when(k >= 2)
        def _():
            pltpu.make_async_copy(dst, p_hbm.at[k - 2, s], sem_o).wait()

        _zero_buf(dst)
        _scan_edges(rows_hbm, cols_hbm, w_hbm, bufs_a, bufs_b,
                    sem_a, sem_b, src, dst)
        pltpu.async_copy(dst, p_hbm.at[k, s], sem_o)

    for slice_i in range(NSLICE // NW):
        s = slice_i * NW + wid
        pltpu.async_copy(x_hbm.at[s], ping, sem_o).wait()

        def pair_body(i, _):
            step(2 * i, s, ping, pong)
            step(2 * i + 1, s, pong, ping)
            return 0

        lax.fori_loop(0, (K - 1) // 2, pair_body, 0)
        step(K - 1, s, ping, pong)

        for k in (K - 2, K - 1):
            dst = pong if k % 2 == 0 else ping
            pltpu.make_async_copy(dst, p_hbm.at[k, s], sem_o).wait()


def _cchain_body(f_ref, out_ref, base_ref, c_ref):
    k = pl.program_id(0)

    @pl.when(k == 0)
    def _():
        ff = lax.dot_general(f_ref[...], f_ref[...],
                             (((0,), (0,)), ((), ())),
                             preferred_element_type=jnp.float32)
        nrm = jnp.sqrt(jnp.sum(ff * ff))
        base_ref[...] = (GAMMA / (nrm + EPS_F)) * ff
        c_ref[...] = base_ref[...]

    @pl.when(k > 0)
    def _():
        c_ref[...] = jnp.dot(c_ref[...], base_ref[...],
                             preferred_element_type=jnp.float32)

    out_ref[0] = c_ref[...]


def _cchain(F):
    return pl.pallas_call(
        _cchain_body,
        grid=(K,),
        in_specs=[pl.BlockSpec((M, M), lambda k: (0, 0))],
        out_specs=pl.BlockSpec((1, M, M), lambda k: (k, 0, 0)),
        out_shape=jax.ShapeDtypeStruct((K, M, M), jnp.float32),
        scratch_shapes=[pltpu.VMEM((M, M), jnp.float32),
                        pltpu.VMEM((M, M), jnp.float32)],
    )(F)


BN = 2048


def _accum_body(x_ref, p_ref, c_ref, out_ref, acc_ref):
    k = pl.program_id(1)

    @pl.when(k == 0)
    def _():
        acc_ref[...] = x_ref[...]

    @pl.when(k > 0)
    def _():
        acc_ref[...] += lax.dot_general(p_ref[0], c_ref[0],
                                        (((0,), (0,)), ((), ())),
                                        preferred_element_type=jnp.float32)

    @pl.when(k == MAX_ITER - 1)
    def _():
        out_ref[...] = acc_ref[...]


def _accumulate(Xp, Pt, C):
    nb = NPAD // BN
    return pl.pallas_call(
        _accum_body,
        grid=(nb, MAX_ITER),
        in_specs=[
            pl.BlockSpec((BN, M), lambda b, k: (b, 0)),
            pl.BlockSpec((1, M, BN), lambda b, k: (jnp.maximum(k - 1, 0), 0, b)),
            pl.BlockSpec((1, M, M), lambda b, k: (jnp.maximum(k - 1, 0), 0, 0)),
        ],
        out_specs=pl.BlockSpec((BN, M), lambda b, k: (b, 0)),
        out_shape=jax.ShapeDtypeStruct((NPAD, M), jnp.float32),
        scratch_shapes=[pltpu.VMEM((BN, M), jnp.float32)],
    )(Xp, Pt, C)


def kernel(X, edge_index, edge_weight, F):
    rows = edge_index[0].astype(jnp.int32)
    cols = edge_index[1].astype(jnp.int32)
    w = edge_weight.astype(jnp.float32)

    Xp = jnp.pad(X, ((0, NPAD - N), (0, 0)))
    x_sl = Xp.reshape(NPAD, NSLICE, FPW).transpose(1, 2, 0).reshape(NSLICE, SLICE_WORDS)
    p_sl = _sc_propagate(x_sl, rows, cols, w)
    Pt = p_sl.reshape(K, M, NPAD)

    C = _cchain(F)
    return _accumulate(Xp, Pt, C)[:N]


# speedup vs baseline: 5.4419x; 1.0583x over previous
import functools

import jax
import jax.numpy as jnp
from jax import lax
from jax.experimental import pallas as pl
from jax.experimental.pallas import tpu as pltpu
from jax.experimental.pallas import tpu_sc as plsc

N = 10000
E = 160000
M = 256
GAMMA = 0.8
MAX_ITER = 30
EPS_F = 1e-12

K = MAX_ITER - 1
NC, NS, LANES = 2, 16, 16
NW = NC * NS
FPW = 4
NSLICE = M // FPW
NPAD = 10240
SLICE_WORDS = NPAD * FPW
CHUNK = 8000
NCHUNK = E // CHUNK
CGROUPS = CHUNK // LANES
UNROLL = 10


def _sc_mesh():
    return plsc.VectorSubcoreMesh(core_axis_name="c", subcore_axis_name="s",
                                  num_cores=NC, num_subcores=NS)


def _issue_chunk(ci, rc_hbm, w_hbm, rcb, wb, sem):
    off = ci * CHUNK
    pltpu.async_copy(rc_hbm.at[pl.ds(off, CHUNK)], rcb, sem)
    pltpu.async_copy(w_hbm.at[pl.ds(off, CHUNK)], wb, sem)


def _wait_chunk(ci, rc_hbm, w_hbm, rcb, wb, sem):
    off = ci * CHUNK
    pltpu.make_async_copy(rc_hbm.at[pl.ds(off, CHUNK)], rcb, sem).wait()
    pltpu.make_async_copy(w_hbm.at[pl.ds(off, CHUNK)], wb, sem).wait()


def _process_chunk(rcb, wb, src, dst):
    @plsc.parallel_loop(0, CGROUPS, unroll=UNROLL)
    def _group(gi):
        b0 = gi * LANES
        rc = rcb[pl.ds(b0, LANES)]
        w = wb[pl.ds(b0, LANES)]
        r = rc & 0xFFFF
        c = lax.shift_right_logical(rc, 16)
        for j in range(FPW):
            rj = r if j == 0 else r + (j * NPAD)
            cj = c if j == 0 else c + (j * NPAD)
            zj = plsc.load_gather(src, [rj])
            plsc.addupdate_scatter(dst, [cj], zj * w)


def _zero_buf(buf):
    zeros = jnp.zeros((LANES,), jnp.float32)

    @plsc.parallel_loop(0, SLICE_WORDS // LANES, unroll=UNROLL)
    def _z(i):
        buf[pl.ds(i * LANES, LANES)] = zeros


def _scan_edges(rc_hbm, w_hbm, bufs_a, bufs_b, sem_a, sem_b, src, dst):
    _issue_chunk(0, rc_hbm, w_hbm, *bufs_a, sem_a)

    def pair_body(i, _):
        cbase = i * 2
        for b, (bufs, sem, nbufs, nsem) in enumerate(
                ((bufs_a, sem_a, bufs_b, sem_b),
                 (bufs_b, sem_b, bufs_a, sem_a))):
            ci = cbase + b
            _wait_chunk(ci, rc_hbm, w_hbm, *bufs, sem)

            @pl.when(ci + 1 < NCHUNK)
            def _():
                _issue_chunk(ci + 1, rc_hbm, w_hbm, *nbufs, nsem)

            _process_chunk(*bufs, src, dst)
        return 0

    lax.fori_loop(0, NCHUNK // 2, pair_body, 0)


@functools.partial(
    pl.kernel,
    out_type=jax.ShapeDtypeStruct((K, NSLICE, SLICE_WORDS), jnp.float32),
    mesh=_sc_mesh(),
    compiler_params=pltpu.CompilerParams(needs_layout_passes=False),
    scratch_types=[
        pltpu.VMEM((SLICE_WORDS,), jnp.float32),
        pltpu.VMEM((SLICE_WORDS,), jnp.float32),
        pltpu.VMEM((CHUNK,), jnp.int32),
        pltpu.VMEM((CHUNK,), jnp.float32),
        pltpu.VMEM((CHUNK,), jnp.int32),
        pltpu.VMEM((CHUNK,), jnp.float32),
        pltpu.SemaphoreType.DMA,
        pltpu.SemaphoreType.DMA,
        pltpu.SemaphoreType.DMA,
    ],
)
def _sc_propagate(x_hbm, rc_hbm, w_hbm, p_hbm,
                  ping, pong, rca, wa, rcb, wb, sem_a, sem_b, sem_o):
    wid = lax.axis_index("s") * NC + lax.axis_index("c")
    bufs_a = (rca, wa)
    bufs_b = (rcb, wb)

    def step(k, s, src, dst):
        @pl.when(k >= 2)
        def _():
            pltpu.make_async_copy(dst, p_hbm.at[k - 2, s], sem_o).wait()

        _zero_buf(dst)
        _scan_edges(rc_hbm, w_hbm, bufs_a, bufs_b, sem_a, sem_b, src, dst)
        pltpu.async_copy(dst, p_hbm.at[k, s], sem_o)

    for slice_i in range(NSLICE // NW):
        s = slice_i * NW + wid
        pltpu.async_copy(x_hbm.at[s], ping, sem_o).wait()

        def pair_body(i, _):
            step(2 * i, s, ping, pong)
            step(2 * i + 1, s, pong, ping)
            return 0

        lax.fori_loop(0, (K - 1) // 2, pair_body, 0)
        step(K - 1, s, ping, pong)

        for k in (K - 2, K - 1):
            dst = pong if k % 2 == 0 else ping
            pltpu.make_async_copy(dst, p_hbm.at[k, s], sem_o).wait()


def _cchain_body(f_ref, out_ref, base_ref, c_ref):
    k = pl.program_id(0)

    @pl.when(k == 0)
    def _():
        ff = lax.dot_general(f_ref[...], f_ref[...],
                             (((0,), (0,)), ((), ())),
                             preferred_element_type=jnp.float32)
        nrm = jnp.sqrt(jnp.sum(ff * ff))
        base_ref[...] = (GAMMA / (nrm + EPS_F)) * ff
        c_ref[...] = base_ref[...]

    @pl.when(k > 0)
    def _():
        c_ref[...] = jnp.dot(c_ref[...], base_ref[...],
                             preferred_element_type=jnp.float32)

    out_ref[0] = c_ref[...]


def _cchain(F):
    return pl.pallas_call(
        _cchain_body,
        grid=(K,),
        in_specs=[pl.BlockSpec((M, M), lambda k: (0, 0))],
        out_specs=pl.BlockSpec((1, M, M), lambda k: (k, 0, 0)),
        out_shape=jax.ShapeDtypeStruct((K, M, M), jnp.float32),
        scratch_shapes=[pltpu.VMEM((M, M), jnp.float32),
                        pltpu.VMEM((M, M), jnp.float32)],
    )(F)


BN = 2048


def _accum_body(x_ref, p_ref, c_ref, out_ref, acc_ref):
    k = pl.program_id(1)

    @pl.when(k == 0)
    def _():
        acc_ref[...] = x_ref[...]

    @pl.when(k > 0)
    def _():
        acc_ref[...] += lax.dot_general(p_ref[0], c_ref[0],
                                        (((0,), (0,)), ((), ())),
                                        preferred_element_type=jnp.float32)

    @pl.when(k == MAX_ITER - 1)
    def _():
        out_ref[...] = acc_ref[...]


def _accumulate(Xp, Pt, C):
    nb = NPAD // BN
    return pl.pallas_call(
        _accum_body,
        grid=(nb, MAX_ITER),
        in_specs=[
            pl.BlockSpec((BN, M), lambda b, k: (b, 0)),
            pl.BlockSpec((1, M, BN), lambda b, k: (jnp.maximum(k - 1, 0), 0, b)),
            pl.BlockSpec((1, M, M), lambda b, k: (jnp.maximum(k - 1, 0), 0, 0)),
        ],
        out_specs=pl.BlockSpec((BN, M), lambda b, k: (b, 0)),
        out_shape=jax.ShapeDtypeStruct((NPAD, M), jnp.float32),
        scratch_shapes=[pltpu.VMEM((BN, M), jnp.float32)],
    )(Xp, Pt, C)


def kernel(X, edge_index, edge_weight, F):
    rows = edge_index[0].astype(jnp.int32)
    cols = edge_index[1].astype(jnp.int32)
    rc = rows | (cols << 16)
    w = edge_weight.astype(jnp.float32)

    Xp = jnp.pad(X, ((0, NPAD - N), (0, 0)))
    x_sl = Xp.reshape(NPAD, NSLICE, FPW).transpose(1, 2, 0).reshape(NSLICE, SLICE_WORDS)
    p_sl = _sc_propagate(x_sl, rc, w)
    Pt = p_sl.reshape(K, M, NPAD)

    C = _cchain(F)
    return _accumulate(Xp, Pt, C)[:N]


# speedup vs baseline: 5.5244x; 1.0152x over previous
import functools

import jax
import jax.numpy as jnp
from jax import lax
from jax.experimental import pallas as pl
from jax.experimental.pallas import tpu as pltpu
from jax.experimental.pallas import tpu_sc as plsc

N = 10000
E = 160000
M = 256
GAMMA = 0.8
MAX_ITER = 30
EPS_F = 1e-12

K = MAX_ITER - 1
NC, NS, LANES = 2, 16, 16
NW = NC * NS
FPW = 4
NSLICE = M // FPW
NPAD = 10240
SLICE_WORDS = NPAD * FPW
CHUNK = 8000
NCHUNK = E // CHUNK
CGROUPS = CHUNK // LANES
UNROLL = 10


def _sc_mesh():
    return plsc.VectorSubcoreMesh(core_axis_name="c", subcore_axis_name="s",
                                  num_cores=NC, num_subcores=NS)


def _issue_chunk(ci, rc_hbm, w_hbm, rcb, wb, sem):
    off = ci * CHUNK
    pltpu.async_copy(rc_hbm.at[pl.ds(off, CHUNK)], rcb, sem)
    pltpu.async_copy(w_hbm.at[pl.ds(off, CHUNK)], wb, sem)


def _wait_chunk(ci, rc_hbm, w_hbm, rcb, wb, sem):
    off = ci * CHUNK
    pltpu.make_async_copy(rc_hbm.at[pl.ds(off, CHUNK)], rcb, sem).wait()
    pltpu.make_async_copy(w_hbm.at[pl.ds(off, CHUNK)], wb, sem).wait()


def _process_chunk(rcb, wb, src, dst):
    @plsc.parallel_loop(0, CGROUPS, unroll=UNROLL)
    def _group(gi):
        b0 = gi * LANES
        rc = rcb[pl.ds(b0, LANES)]
        w = wb[pl.ds(b0, LANES)]
        r = rc & 0xFFFF
        c = lax.shift_right_logical(rc, 16)
        for j in range(FPW):
            rj = r if j == 0 else r + (j * NPAD)
            cj = c if j == 0 else c + (j * NPAD)
            zj = plsc.load_gather(src, [rj])
            plsc.addupdate_scatter(dst, [cj], zj * w)


def _zero_buf(buf):
    zeros = jnp.zeros((LANES,), jnp.float32)

    @plsc.parallel_loop(0, SLICE_WORDS // LANES, unroll=UNROLL)
    def _z(i):
        buf[pl.ds(i * LANES, LANES)] = zeros


def _scan_edges(rc_hbm, w_hbm, bufs_a, bufs_b, sem_a, sem_b, src, dst):
    def pair_body(i, _):
        cbase = i * 2
        for b, (bufs, sem, nbufs, nsem) in enumerate(
                ((bufs_a, sem_a, bufs_b, sem_b),
                 (bufs_b, sem_b, bufs_a, sem_a))):
            ci = cbase + b
            _wait_chunk(ci, rc_hbm, w_hbm, *bufs, sem)

            @pl.when(ci + 1 < NCHUNK)
            def _():
                _issue_chunk(ci + 1, rc_hbm, w_hbm, *nbufs, nsem)

            _process_chunk(*bufs, src, dst)
        return 0

    lax.fori_loop(0, NCHUNK // 2, pair_body, 0)


@functools.partial(
    pl.kernel,
    out_type=jax.ShapeDtypeStruct((K, NSLICE, SLICE_WORDS), jnp.float32),
    mesh=_sc_mesh(),
    compiler_params=pltpu.CompilerParams(needs_layout_passes=False),
    scratch_types=[
        pltpu.VMEM((SLICE_WORDS,), jnp.float32),
        pltpu.VMEM((SLICE_WORDS,), jnp.float32),
        pltpu.VMEM((CHUNK,), jnp.int32),
        pltpu.VMEM((CHUNK,), jnp.float32),
        pltpu.VMEM((CHUNK,), jnp.int32),
        pltpu.VMEM((CHUNK,), jnp.float32),
        pltpu.SemaphoreType.DMA,
        pltpu.SemaphoreType.DMA,
        pltpu.SemaphoreType.DMA,
    ],
)
def _sc_propagate(x_hbm, rc_hbm, w_hbm, p_hbm,
                  ping, pong, rca, wa, rcb, wb, sem_a, sem_b, sem_o):
    wid = lax.axis_index("s") * NC + lax.axis_index("c")
    bufs_a = (rca, wa)
    bufs_b = (rcb, wb)

    def step(k, s, src, dst):
        @pl.when(k >= 2)
        def _():
            pltpu.make_async_copy(dst, p_hbm.at[k - 2, s], sem_o).wait()

        _issue_chunk(0, rc_hbm, w_hbm, *bufs_a, sem_a)
        _zero_buf(dst)
        _scan_edges(rc_hbm, w_hbm, bufs_a, bufs_b, sem_a, sem_b, src, dst)
        pltpu.async_copy(dst, p_hbm.at[k, s], sem_o)

    for slice_i in range(NSLICE // NW):
        s = slice_i * NW + wid
        pltpu.async_copy(x_hbm.at[s], ping, sem_o).wait()

        def pair_body(i, _):
            step(2 * i, s, ping, pong)
            step(2 * i + 1, s, pong, ping)
            return 0

        lax.fori_loop(0, (K - 1) // 2, pair_body, 0)
        step(K - 1, s, ping, pong)

        for k in (K - 2, K - 1):
            dst = pong if k % 2 == 0 else ping
            pltpu.make_async_copy(dst, p_hbm.at[k, s], sem_o).wait()


def _cchain_body(f_ref, out_ref, base_ref, c_ref):
    k = pl.program_id(0)

    @pl.when(k == 0)
    def _():
        ff = lax.dot_general(f_ref[...], f_ref[...],
                             (((0,), (0,)), ((), ())),
                             preferred_element_type=jnp.float32)
        nrm = jnp.sqrt(jnp.sum(ff * ff))
        base_ref[...] = (GAMMA / (nrm + EPS_F)) * ff
        c_ref[...] = base_ref[...]

    @pl.when(k > 0)
    def _():
        c_ref[...] = jnp.dot(c_ref[...], base_ref[...],
                             preferred_element_type=jnp.float32)

    out_ref[0] = c_ref[...]


def _cchain(F):
    return pl.pallas_call(
        _cchain_body,
        grid=(K,),
        in_specs=[pl.BlockSpec((M, M), lambda k: (0, 0))],
        out_specs=pl.BlockSpec((1, M, M), lambda k: (k, 0, 0)),
        out_shape=jax.ShapeDtypeStruct((K, M, M), jnp.float32),
        scratch_shapes=[pltpu.VMEM((M, M), jnp.float32),
                        pltpu.VMEM((M, M), jnp.float32)],
    )(F)


BN = 2048


def _accum_body(x_ref, p_ref, c_ref, out_ref, acc_ref):
    k = pl.program_id(1)

    @pl.when(k == 0)
    def _():
        acc_ref[...] = x_ref[...]

    @pl.when(k > 0)
    def _():
        acc_ref[...] += lax.dot_general(p_ref[0], c_ref[0],
                                        (((0,), (0,)), ((), ())),
                                        preferred_element_type=jnp.float32)

    @pl.when(k == MAX_ITER - 1)
    def _():
        out_ref[...] = acc_ref[...]


def _accumulate(Xp, Pt, C):
    nb = NPAD // BN
    return pl.pallas_call(
        _accum_body,
        grid=(nb, MAX_ITER),
        in_specs=[
            pl.BlockSpec((BN, M), lambda b, k: (b, 0)),
            pl.BlockSpec((1, M, BN), lambda b, k: (jnp.maximum(k - 1, 0), 0, b)),
            pl.BlockSpec((1, M, M), lambda b, k: (jnp.maximum(k - 1, 0), 0, 0)),
        ],
        out_specs=pl.BlockSpec((BN, M), lambda b, k: (b, 0)),
        out_shape=jax.ShapeDtypeStruct((NPAD, M), jnp.float32),
        scratch_shapes=[pltpu.VMEM((BN, M), jnp.float32)],
    )(Xp, Pt, C)


def kernel(X, edge_index, edge_weight, F):
    rows = edge_index[0].astype(jnp.int32)
    cols = edge_index[1].astype(jnp.int32)
    rc = rows | (cols << 16)
    w = edge_weight.astype(jnp.float32)

    Xp = jnp.pad(X, ((0, NPAD - N), (0, 0)))
    x_sl = Xp.reshape(NPAD, NSLICE, FPW).transpose(1, 2, 0).reshape(NSLICE, SLICE_WORDS)
    p_sl = _sc_propagate(x_sl, rc, w)
    Pt = p_sl.reshape(K, M, NPAD)

    C = _cchain(F)
    return _accumulate(Xp, Pt, C)[:N]
